# baseline (speedup 1.0000x reference)
import functools

import jax
import jax.numpy as jnp
from jax import lax
from jax.experimental import pallas as pl
from jax.experimental.pallas import tpu as pltpu
from jax.experimental.pallas import tpu_sc as plsc

L = 16
BE = 128
CH = 5
BR = 800


@functools.lru_cache(maxsize=None)
def _make_sc_gather(N, Ec, D):
    assert Ec % BE == 0
    NB = Ec // BE
    info = plsc.get_sparse_core_info()
    NC = info.num_cores
    NW = NC * info.num_subcores
    T = (NB + NW - 1) // NW
    T2 = T + (T % 2)
    mesh = plsc.VectorSubcoreMesh(core_axis_name="c", subcore_axis_name="s")

    @functools.partial(
        pl.kernel,
        out_type=jax.ShapeDtypeStruct((Ec, D), jnp.float32),
        mesh=mesh,
        scratch_types=[
            pltpu.VMEM((T2 * BE,), jnp.int32),
            pltpu.VMEM((2, BE, D), jnp.float32),
            pltpu.SemaphoreType.DMA,
            pltpu.SemaphoreType.DMA,
            pltpu.SemaphoreType.DMA,
            pltpu.SemaphoreType.DMA,
            pltpu.SemaphoreType.DMA,
        ],
    )
    def body(x_hbm, idx_hbm, out_hbm, idx_all, rows, isem, gsem0, gsem1,
             osem0, osem1):
        wid = lax.axis_index("s") * NC + lax.axis_index("c")
        gsem = (gsem0, gsem1)
        osem = (osem0, osem1)

        def valid(t):
            return wid + t * NW < NB

        def base_of(t):
            return (wid + t * NW) * BE

        @pl.loop(0, T2)
        def _fire(t):
            @pl.when(valid(t))
            def _():
                pltpu.async_copy(idx_hbm.at[pl.ds(base_of(t), BE)],
                                 idx_all.at[pl.ds(t * BE, BE)], isem)

        @pl.loop(0, T2)
        def _drain(t):
            @pl.when(valid(t))
            def _():
                pltpu.make_async_copy(idx_hbm.at[pl.ds(base_of(t), BE)],
                                      idx_all.at[pl.ds(t * BE, BE)],
                                      isem).wait()
                for j in range(BE // L):
                    ix = pl.ds(t * BE + j * L, L)
                    idx_all[ix] = jnp.maximum(idx_all[ix], 0)

        def fire_gather(t, buf):
            pltpu.async_copy(x_hbm.at[idx_all.at[pl.ds(t * BE, BE)]],
                             rows.at[buf], gsem[buf])

        def drain_out(buf):
            pltpu.make_async_copy(rows.at[buf], out_hbm.at[pl.ds(0, BE), :],
                                  osem[buf]).wait()

        @pl.when(valid(0))
        def _():
            fire_gather(0, 0)

        @pl.loop(0, T2, step=2)
        def _steps(i):
            for b in (0, 1):
                t = i + b

                @pl.when((t + 1 < T2) & valid(t + 1))
                def _():
                    @pl.when(t >= 1)
                    def _():
                        drain_out(1 - b)
                    fire_gather(t + 1, 1 - b)

                @pl.when(valid(t))
                def _():
                    pltpu.make_async_copy(
                        x_hbm.at[idx_all.at[pl.ds(t * BE, BE)]],
                        rows.at[b], gsem[b]).wait()
                    pltpu.async_copy(rows.at[b],
                                     out_hbm.at[pl.ds(base_of(t), BE), :],
                                     osem[b])

        assert NB >= 2 * NW
        for b in (0, 1):
            drain_out(b)

    return body


@functools.lru_cache(maxsize=None)
def _make_tc_scale(E, Ec, D, off_rows):
    assert Ec % BR == 0 and off_rows % BR == 0
    nblk = Ec // BR
    off = off_rows // BR

    def tc_body(prev_ref, g_ref, geo_ref, idx_ref, out_ref):
        del prev_ref
        g = g_ref[...]
        mask = (idx_ref[...] != -1)
        scale = mask.astype(jnp.float32) / (geo_ref[:, 0:1] + 1e-6)
        sp = geo_ref[:, 1:2]
        cp = geo_ref[:, 2:3]
        st = geo_ref[:, 3:4]
        ct = geo_ref[:, 4:5]
        out_ref[:, 0:D] = g * (st * cp * scale)
        out_ref[:, D:2 * D] = g * (st * sp * scale)
        out_ref[:, 2 * D:3 * D] = g * (ct * scale)

    first = off_rows == 0

    def tc_first(g_ref, geo_ref, idx_ref, out_ref):
        tc_body(None, g_ref, geo_ref, idx_ref, out_ref)

    data_specs = [
        pl.BlockSpec((BR, D), lambda i: (i, 0)),
        pl.BlockSpec((BR, 5), lambda i: (i, 0)),
        pl.BlockSpec((BR, 1), lambda i: (i, 0)),
    ]
    return pl.pallas_call(
        tc_first if first else tc_body,
        grid=(nblk,),
        in_specs=data_specs if first else (
            [pl.BlockSpec(memory_space=pl.ANY)] + data_specs),
        out_specs=pl.BlockSpec((BR, 3 * D), lambda i: (i + off, 0)),
        out_shape=jax.ShapeDtypeStruct((E, 3 * D), jnp.float32),
        input_output_aliases={} if first else {0: 0},
    )


def kernel(x, geo_features, neighbor_indices):
    N, D = x.shape
    _, K, _ = geo_features.shape
    E = N * K
    Ec = E // CH
    idx = neighbor_indices.reshape(E).astype(jnp.int32)
    geo = geo_features.reshape(E, 5)
    idxcol = idx.reshape(E, 1)

    sc_gather = _make_sc_gather(N, Ec, D)
    gathered = [sc_gather(x, idx[c * Ec:(c + 1) * Ec]) for c in range(CH)]

    out = _make_tc_scale(E, Ec, D, 0)(
        gathered[0], geo[0:Ec], idxcol[0:Ec])
    for c in range(1, CH):
        out = _make_tc_scale(E, Ec, D, c * Ec)(
            out, gathered[c], geo[c * Ec:(c + 1) * Ec],
            idxcol[c * Ec:(c + 1) * Ec])
    return out.reshape(N, K, 3 * D)


# speedup vs baseline: 1.3781x; 1.3781x over previous
import functools

import jax
import jax.numpy as jnp
from jax import lax
from jax.experimental import pallas as pl
from jax.experimental.pallas import tpu as pltpu
from jax.experimental.pallas import tpu_sc as plsc

L = 16
BE = 128
CH = 5
BR = 640


@functools.lru_cache(maxsize=None)
def _make_sc_gather(N, Ec, D):
    assert Ec % BE == 0
    NB = Ec // BE
    info = plsc.get_sparse_core_info()
    NC = info.num_cores
    NW = NC * info.num_subcores
    T = (NB + NW - 1) // NW
    T2 = T + (T % 2)
    mesh = plsc.VectorSubcoreMesh(core_axis_name="c", subcore_axis_name="s")

    @functools.partial(
        pl.kernel,
        out_type=jax.ShapeDtypeStruct((Ec, D), jnp.float32),
        mesh=mesh,
        scratch_types=[
            pltpu.VMEM((T2 * BE,), jnp.int32),
            pltpu.VMEM((2, BE, D), jnp.float32),
            pltpu.SemaphoreType.DMA,
            pltpu.SemaphoreType.DMA,
            pltpu.SemaphoreType.DMA,
            pltpu.SemaphoreType.DMA,
            pltpu.SemaphoreType.DMA,
        ],
    )
    def body(x_hbm, idx_hbm, out_hbm, idx_all, rows, isem, gsem0, gsem1,
             osem0, osem1):
        wid = lax.axis_index("s") * NC + lax.axis_index("c")
        gsem = (gsem0, gsem1)
        osem = (osem0, osem1)

        def valid(t):
            return wid + t * NW < NB

        def base_of(t):
            return (wid + t * NW) * BE

        @pl.loop(0, T2)
        def _fire(t):
            @pl.when(valid(t))
            def _():
                pltpu.async_copy(idx_hbm.at[pl.ds(base_of(t), BE)],
                                 idx_all.at[pl.ds(t * BE, BE)], isem)

        @pl.loop(0, T2)
        def _drain(t):
            @pl.when(valid(t))
            def _():
                pltpu.make_async_copy(idx_hbm.at[pl.ds(base_of(t), BE)],
                                      idx_all.at[pl.ds(t * BE, BE)],
                                      isem).wait()
                for j in range(BE // L):
                    ix = pl.ds(t * BE + j * L, L)
                    idx_all[ix] = jnp.maximum(idx_all[ix], 0)

        def fire_gather(t, buf):
            pltpu.async_copy(x_hbm.at[idx_all.at[pl.ds(t * BE, BE)]],
                             rows.at[buf], gsem[buf])

        def drain_out(buf):
            pltpu.make_async_copy(rows.at[buf], out_hbm.at[pl.ds(0, BE), :],
                                  osem[buf]).wait()

        @pl.when(valid(0))
        def _():
            fire_gather(0, 0)

        @pl.loop(0, T2, step=2)
        def _steps(i):
            for b in (0, 1):
                t = i + b

                @pl.when((t + 1 < T2) & valid(t + 1))
                def _():
                    @pl.when(t >= 1)
                    def _():
                        drain_out(1 - b)
                    fire_gather(t + 1, 1 - b)

                @pl.when(valid(t))
                def _():
                    pltpu.make_async_copy(
                        x_hbm.at[idx_all.at[pl.ds(t * BE, BE)]],
                        rows.at[b], gsem[b]).wait()
                    pltpu.async_copy(rows.at[b],
                                     out_hbm.at[pl.ds(base_of(t), BE), :],
                                     osem[b])

        assert NB >= 2 * NW
        for b in (0, 1):
            drain_out(b)

    return body


@functools.lru_cache(maxsize=None)
def _make_tc_scale(E, Ec, D, off_rows):
    assert Ec % BR == 0 and off_rows % BR == 0
    nblk = Ec // BR
    off = off_rows // BR

    def tc_body(prev_ref, g_ref, s8_ref, out_ref):
        del prev_ref
        g = g_ref[...]
        t8 = jnp.transpose(s8_ref[...])
        mask = (t8[:, 5:6] != -1.0)
        scale = mask.astype(jnp.float32) / (t8[:, 0:1] + 1e-6)
        sp = t8[:, 1:2]
        cp = t8[:, 2:3]
        st = t8[:, 3:4]
        ct = t8[:, 4:5]
        out_ref[:, 0:D] = g * (st * cp * scale)
        out_ref[:, D:2 * D] = g * (st * sp * scale)
        out_ref[:, 2 * D:3 * D] = g * (ct * scale)

    first = off_rows == 0

    def tc_first(g_ref, s8_ref, out_ref):
        tc_body(None, g_ref, s8_ref, out_ref)

    data_specs = [
        pl.BlockSpec((BR, D), lambda i: (i, 0)),
        pl.BlockSpec((8, BR), lambda i: (0, i)),
    ]
    return pl.pallas_call(
        tc_first if first else tc_body,
        grid=(nblk,),
        in_specs=data_specs if first else (
            [pl.BlockSpec(memory_space=pl.ANY)] + data_specs),
        out_specs=pl.BlockSpec((BR, 3 * D), lambda i: (i + off, 0)),
        out_shape=jax.ShapeDtypeStruct((E, 3 * D), jnp.float32),
        input_output_aliases={} if first else {0: 0},
    )


def kernel(x, geo_features, neighbor_indices):
    N, D = x.shape
    _, K, _ = geo_features.shape
    E = N * K
    Ec = E // CH
    idx = neighbor_indices.reshape(E).astype(jnp.int32)
    s8 = jnp.concatenate([
        geo_features.reshape(E, 5).T,
        idx.astype(jnp.float32)[None, :],
        jnp.zeros((2, E), jnp.float32),
    ], axis=0)

    sc_gather = _make_sc_gather(N, Ec, D)
    gathered = [sc_gather(x, idx[c * Ec:(c + 1) * Ec]) for c in range(CH)]

    out = _make_tc_scale(E, Ec, D, 0)(gathered[0], s8[:, 0:Ec])
    for c in range(1, CH):
        out = _make_tc_scale(E, Ec, D, c * Ec)(
            out, gathered[c], s8[:, c * Ec:(c + 1) * Ec])
    return out.reshape(N, K, 3 * D)


# speedup vs baseline: 1.6082x; 1.1669x over previous
import functools

import jax
import jax.numpy as jnp
from jax import lax
from jax.experimental import pallas as pl
from jax.experimental.pallas import tpu as pltpu
from jax.experimental.pallas import tpu_sc as plsc

L = 16
BE = 128
CH = 5
BR = 1280


@functools.lru_cache(maxsize=None)
def _make_sc_gather(N, Ec, D):
    assert Ec % BE == 0
    NB = Ec // BE
    info = plsc.get_sparse_core_info()
    NC = info.num_cores
    NW = NC * info.num_subcores
    T = (NB + NW - 1) // NW
    T2 = T + (T % 2)
    mesh = plsc.VectorSubcoreMesh(core_axis_name="c", subcore_axis_name="s")

    @functools.partial(
        pl.kernel,
        out_type=jax.ShapeDtypeStruct((Ec, D), jnp.float32),
        mesh=mesh,
        scratch_types=[
            pltpu.VMEM((T2 * BE,), jnp.int32),
            pltpu.VMEM((2, BE, D), jnp.float32),
            pltpu.SemaphoreType.DMA,
            pltpu.SemaphoreType.DMA,
            pltpu.SemaphoreType.DMA,
            pltpu.SemaphoreType.DMA,
            pltpu.SemaphoreType.DMA,
        ],
    )
    def body(x_hbm, idx_hbm, out_hbm, idx_all, rows, isem, gsem0, gsem1,
             osem0, osem1):
        wid = lax.axis_index("s") * NC + lax.axis_index("c")
        gsem = (gsem0, gsem1)
        osem = (osem0, osem1)

        def valid(t):
            return wid + t * NW < NB

        def base_of(t):
            return (wid + t * NW) * BE

        @pl.loop(0, T2)
        def _fire(t):
            @pl.when(valid(t))
            def _():
                pltpu.async_copy(idx_hbm.at[pl.ds(base_of(t), BE)],
                                 idx_all.at[pl.ds(t * BE, BE)], isem)

        @pl.loop(0, T2)
        def _drain(t):
            @pl.when(valid(t))
            def _():
                pltpu.make_async_copy(idx_hbm.at[pl.ds(base_of(t), BE)],
                                      idx_all.at[pl.ds(t * BE, BE)],
                                      isem).wait()
                for j in range(BE // L):
                    ix = pl.ds(t * BE + j * L, L)
                    idx_all[ix] = jnp.maximum(idx_all[ix], 0)

        def fire_gather(t, buf):
            pltpu.async_copy(x_hbm.at[idx_all.at[pl.ds(t * BE, BE)]],
                             rows.at[buf], gsem[buf])

        def drain_out(buf):
            pltpu.make_async_copy(rows.at[buf], out_hbm.at[pl.ds(0, BE), :],
                                  osem[buf]).wait()

        @pl.when(valid(0))
        def _():
            fire_gather(0, 0)

        @pl.loop(0, T2, step=2)
        def _steps(i):
            for b in (0, 1):
                t = i + b

                @pl.when((t + 1 < T2) & valid(t + 1))
                def _():
                    @pl.when(t >= 1)
                    def _():
                        drain_out(1 - b)
                    fire_gather(t + 1, 1 - b)

                @pl.when(valid(t))
                def _():
                    pltpu.make_async_copy(
                        x_hbm.at[idx_all.at[pl.ds(t * BE, BE)]],
                        rows.at[b], gsem[b]).wait()
                    pltpu.async_copy(rows.at[b],
                                     out_hbm.at[pl.ds(base_of(t), BE), :],
                                     osem[b])

        assert NB >= 2 * NW
        for b in (0, 1):
            drain_out(b)

    return body


@functools.lru_cache(maxsize=None)
def _make_tc_scale(E, Ec, D, off_rows):
    assert Ec % BR == 0 and off_rows % BR == 0
    nblk = Ec // BR
    off = off_rows // BR

    def tc_body(prev_ref, g_ref, s8_ref, out_ref):
        del prev_ref
        g = g_ref[...]
        t8 = jnp.transpose(s8_ref[...])
        mask = (t8[:, 5:6] != -1.0)
        scale = mask.astype(jnp.float32) / (t8[:, 0:1] + 1e-6)
        sp = t8[:, 1:2]
        cp = t8[:, 2:3]
        st = t8[:, 3:4]
        ct = t8[:, 4:5]
        out_ref[:, 0:D] = g * (st * cp * scale)
        out_ref[:, D:2 * D] = g * (st * sp * scale)
        out_ref[:, 2 * D:3 * D] = g * (ct * scale)

    first = off_rows == 0

    def tc_first(g_ref, s8_ref, out_ref):
        tc_body(None, g_ref, s8_ref, out_ref)

    data_specs = [
        pl.BlockSpec((BR, D), lambda i: (i, 0)),
        pl.BlockSpec((8, BR), lambda i: (0, i)),
    ]
    return pl.pallas_call(
        tc_first if first else tc_body,
        grid=(nblk,),
        in_specs=data_specs if first else (
            [pl.BlockSpec(memory_space=pl.ANY)] + data_specs),
        out_specs=pl.BlockSpec((BR, 3 * D), lambda i: (i + off, 0)),
        out_shape=jax.ShapeDtypeStruct((E, 3 * D), jnp.float32),
        input_output_aliases={} if first else {0: 0},
    )


def kernel(x, geo_features, neighbor_indices):
    N, D = x.shape
    _, K, _ = geo_features.shape
    E = N * K
    Ec = E // CH
    idx = neighbor_indices.reshape(E).astype(jnp.int32)
    s8 = jnp.concatenate([
        geo_features.reshape(E, 5).T,
        idx.astype(jnp.float32)[None, :],
        jnp.zeros((2, E), jnp.float32),
    ], axis=0)

    sc_gather = _make_sc_gather(N, Ec, D)
    gathered = [sc_gather(x, idx[c * Ec:(c + 1) * Ec]) for c in range(CH)]

    out = _make_tc_scale(E, Ec, D, 0)(gathered[0], s8[:, 0:Ec])
    for c in range(1, CH):
        out = _make_tc_scale(E, Ec, D, c * Ec)(
            out, gathered[c], s8[:, c * Ec:(c + 1) * Ec])
    return out.reshape(N, K, 3 * D)


# speedup vs baseline: 1.6667x; 1.0364x over previous
import functools

import jax
import jax.numpy as jnp
from jax import lax
from jax.experimental import pallas as pl
from jax.experimental.pallas import tpu as pltpu
from jax.experimental.pallas import tpu_sc as plsc

L = 16
BE = 128
CH = 5
BR = 3200


@functools.lru_cache(maxsize=None)
def _make_sc_gather(N, Ec, D):
    assert Ec % BE == 0
    NB = Ec // BE
    info = plsc.get_sparse_core_info()
    NC = info.num_cores
    NW = NC * info.num_subcores
    T = (NB + NW - 1) // NW
    T2 = T + (T % 2)
    mesh = plsc.VectorSubcoreMesh(core_axis_name="c", subcore_axis_name="s")

    @functools.partial(
        pl.kernel,
        out_type=jax.ShapeDtypeStruct((Ec, D), jnp.float32),
        mesh=mesh,
        scratch_types=[
            pltpu.VMEM((T2 * BE,), jnp.int32),
            pltpu.VMEM((2, BE, D), jnp.float32),
            pltpu.SemaphoreType.DMA,
            pltpu.SemaphoreType.DMA,
            pltpu.SemaphoreType.DMA,
            pltpu.SemaphoreType.DMA,
            pltpu.SemaphoreType.DMA,
        ],
    )
    def body(x_hbm, idx_hbm, out_hbm, idx_all, rows, isem, gsem0, gsem1,
             osem0, osem1):
        wid = lax.axis_index("s") * NC + lax.axis_index("c")
        gsem = (gsem0, gsem1)
        osem = (osem0, osem1)

        def valid(t):
            return wid + t * NW < NB

        def base_of(t):
            return (wid + t * NW) * BE

        @pl.loop(0, T2)
        def _fire(t):
            @pl.when(valid(t))
            def _():
                pltpu.async_copy(idx_hbm.at[pl.ds(base_of(t), BE)],
                                 idx_all.at[pl.ds(t * BE, BE)], isem)

        @pl.loop(0, T2)
        def _drain(t):
            @pl.when(valid(t))
            def _():
                pltpu.make_async_copy(idx_hbm.at[pl.ds(base_of(t), BE)],
                                      idx_all.at[pl.ds(t * BE, BE)],
                                      isem).wait()
                for j in range(BE // L):
                    ix = pl.ds(t * BE + j * L, L)
                    idx_all[ix] = jnp.maximum(idx_all[ix], 0)

        def fire_gather(t, buf):
            pltpu.async_copy(x_hbm.at[idx_all.at[pl.ds(t * BE, BE)]],
                             rows.at[buf], gsem[buf])

        def drain_out(buf):
            pltpu.make_async_copy(rows.at[buf], out_hbm.at[pl.ds(0, BE), :],
                                  osem[buf]).wait()

        @pl.when(valid(0))
        def _():
            fire_gather(0, 0)

        @pl.loop(0, T2, step=2)
        def _steps(i):
            for b in (0, 1):
                t = i + b

                @pl.when((t + 1 < T2) & valid(t + 1))
                def _():
                    @pl.when(t >= 1)
                    def _():
                        drain_out(1 - b)
                    fire_gather(t + 1, 1 - b)

                @pl.when(valid(t))
                def _():
                    pltpu.make_async_copy(
                        x_hbm.at[idx_all.at[pl.ds(t * BE, BE)]],
                        rows.at[b], gsem[b]).wait()
                    pltpu.async_copy(rows.at[b],
                                     out_hbm.at[pl.ds(base_of(t), BE), :],
                                     osem[b])

        assert NB >= 2 * NW
        for b in (0, 1):
            drain_out(b)

    return body


@functools.lru_cache(maxsize=None)
def _make_tc_scale(E, Ec, D, off_rows):
    assert Ec % BR == 0 and off_rows % BR == 0
    nblk = Ec // BR
    off = off_rows // BR

    def tc_body(prev_ref, g_ref, s8_ref, out_ref):
        del prev_ref
        g = g_ref[...]
        t8 = jnp.transpose(s8_ref[...])
        mask = (t8[:, 5:6] != -1.0)
        scale = mask.astype(jnp.float32) / (t8[:, 0:1] + 1e-6)
        sp = t8[:, 1:2]
        cp = t8[:, 2:3]
        st = t8[:, 3:4]
        ct = t8[:, 4:5]
        out_ref[:, 0:D] = g * (st * cp * scale)
        out_ref[:, D:2 * D] = g * (st * sp * scale)
        out_ref[:, 2 * D:3 * D] = g * (ct * scale)

    first = off_rows == 0

    def tc_first(g_ref, s8_ref, out_ref):
        tc_body(None, g_ref, s8_ref, out_ref)

    data_specs = [
        pl.BlockSpec((BR, D), lambda i: (i, 0)),
        pl.BlockSpec((8, BR), lambda i: (0, i)),
    ]
    return pl.pallas_call(
        tc_first if first else tc_body,
        grid=(nblk,),
        in_specs=data_specs if first else (
            [pl.BlockSpec(memory_space=pl.ANY)] + data_specs),
        out_specs=pl.BlockSpec((BR, 3 * D), lambda i: (i + off, 0)),
        out_shape=jax.ShapeDtypeStruct((E, 3 * D), jnp.float32),
        input_output_aliases={} if first else {0: 0},
    )


def kernel(x, geo_features, neighbor_indices):
    N, D = x.shape
    _, K, _ = geo_features.shape
    E = N * K
    Ec = E // CH
    idx = neighbor_indices.reshape(E).astype(jnp.int32)
    s8 = jnp.concatenate([
        geo_features.reshape(E, 5).T,
        idx.astype(jnp.float32)[None, :],
        jnp.zeros((2, E), jnp.float32),
    ], axis=0)

    sc_gather = _make_sc_gather(N, Ec, D)
    gathered = [sc_gather(x, idx[c * Ec:(c + 1) * Ec]) for c in range(CH)]

    out = _make_tc_scale(E, Ec, D, 0)(gathered[0], s8[:, 0:Ec])
    for c in range(1, CH):
        out = _make_tc_scale(E, Ec, D, c * Ec)(
            out, gathered[c], s8[:, c * Ec:(c + 1) * Ec])
    return out.reshape(N, K, 3 * D)
